# Initial kernel scaffold; baseline (speedup 1.0000x reference)
import jax
import jax.numpy as jnp
from jax.experimental import pallas as pl


def kernel(x, edge_index, Wl1, bl1, Wr1, Wl2, bl2, Wr2, Wp1, bp1, Wp2, bp2):
    raise NotImplementedError("write your pallas kernel here")


# speedup vs baseline: 5.8846x; 5.8846x over previous
import functools

import jax
import jax.numpy as jnp
from jax import lax
from jax.experimental import pallas as pl
from jax.experimental.pallas import tpu as pltpu
from jax.experimental.pallas import tpu_sc as plsc

NC = 2
NS = 16
LANES = 16
K = 128


@functools.lru_cache(maxsize=None)
def _make_sc_agg(n, d, ch, with_counts):
    dh = d // NC
    zrows = 128 * (-(-n // (128 * NS)))
    sh_rows = zrows * NS
    last = n - (NS - 1) * zrows
    cnt_pad = sh_rows

    mesh = plsc.VectorSubcoreMesh(
        core_axis_name="c", subcore_axis_name="s",
        num_cores=NC, num_subcores=NS)

    out_type = [jax.ShapeDtypeStruct((n, d), jnp.float32)]
    if with_counts:
        out_type.append(jax.ShapeDtypeStruct((cnt_pad,), jnp.float32))

    scratch = [
        pltpu.VMEM((ch, K), jnp.int32),
        pltpu.VMEM((ch, K), jnp.int32),
        pltpu.VMEM((K, dh), jnp.float32),
        pltpu.VMEM((K,), jnp.float32),
        pltpu.VMEM_SHARED((sh_rows, dh), jnp.float32),
        pltpu.VMEM_SHARED((cnt_pad,), jnp.float32),
        pltpu.SemaphoreType.DMA,
    ]

    @functools.partial(pl.kernel, mesh=mesh, out_type=out_type,
                       scratch_types=scratch)
    def sc_agg(x2, srcx, dstp, z2d, z1d, ones_h, *refs):
        if with_counts:
            agg_out, cnt_out = refs[0], refs[1]
            rest = refs[2:]
        else:
            agg_out = refs[0]
            rest = refs[1:]
        idx_s, idx_d, rows, ones_v, agg_sh, cnt_sh, sem = rest

        c = lax.axis_index("c")
        s = lax.axis_index("s")

        pltpu.sync_copy(z2d, agg_sh.at[pl.ds(s * zrows, zrows)])
        if with_counts:
            @pl.when(c == 0)
            def _():
                pltpu.sync_copy(z1d, cnt_sh.at[pl.ds(s * zrows, zrows)])
                pltpu.sync_copy(ones_h, ones_v)

        pltpu.sync_copy(srcx.at[c, s], idx_s)
        pltpu.sync_copy(dstp.at[s], idx_d)
        plsc.subcore_barrier()

        def chunk(j, _):
            pltpu.async_copy(x2.at[idx_s.at[j]], rows, sem).wait()
            pltpu.sync_copy(rows, agg_sh.at[idx_d.at[j]], add=True)
            if with_counts:
                @pl.when(c == 0)
                def _():
                    pltpu.sync_copy(ones_v, cnt_sh.at[idx_d.at[j]], add=True)
            return 0

        lax.fori_loop(0, ch, chunk, 0)
        plsc.subcore_barrier()

        @pl.when(s < NS - 1)
        def _():
            pltpu.sync_copy(agg_sh.at[pl.ds(s * zrows, zrows)],
                            agg_out.at[pl.ds(s * zrows, zrows),
                                       pl.ds(c * dh, dh)])

        @pl.when(s == NS - 1)
        def _():
            pltpu.sync_copy(agg_sh.at[pl.ds((NS - 1) * zrows, last)],
                            agg_out.at[pl.ds((NS - 1) * zrows, last),
                                       pl.ds(c * dh, dh)])
        if with_counts:
            @pl.when(c == 0)
            def _():
                pltpu.sync_copy(cnt_sh.at[pl.ds(s * zrows, zrows)],
                                cnt_out.at[pl.ds(s * zrows, zrows)])

    return sc_agg


def _edge_setup(src, dst, n):
    e = src.shape[0]
    et = e // NS
    ch = -(-et // K)
    pad = ch * K - et
    srct = src.reshape(NS, et)
    dstt = dst.reshape(NS, et)
    if pad:
        psrc = jnp.broadcast_to((jnp.arange(pad, dtype=jnp.int32) * 97) % n,
                                (NS, pad))
        pdst = jnp.broadcast_to(n + (jnp.arange(pad, dtype=jnp.int32) % LANES),
                                (NS, pad))
        srct = jnp.concatenate([srct, psrc], axis=1)
        dstt = jnp.concatenate([dstt, pdst], axis=1)
    srcx = jnp.stack([srct * 2, srct * 2 + 1]).reshape(NC, NS, ch, K)
    dstp = dstt.reshape(NS, ch, K)
    return srcx, dstp, ch


def _dot_t(a, w):
    return lax.dot_general(a, w, (((1,), (1,)), ((), ())),
                           preferred_element_type=jnp.float32)


def _tc_layer_body(agg, cnt, x, wl, bl, wr, out):
    inv = 1.0 / jnp.maximum(cnt[...], 1.0)
    mean = agg[...] * inv
    h = _dot_t(mean, wl[...]) + _dot_t(x[...], wr[...]) + bl[...]
    out[...] = jnp.maximum(h, 0.0)


def _tc_final_body(agg, cnt, x, wl, bl, wr, wp1, bp1, wp2, bp2, out):
    inv = 1.0 / jnp.maximum(cnt[...], 1.0)
    mean = agg[...] * inv
    h = _dot_t(mean, wl[...]) + _dot_t(x[...], wr[...]) + bl[...]
    h = jnp.maximum(h, 0.0)
    h = _dot_t(h, wp1[...]) + bp1[...]
    out[...] = _dot_t(h, wp2[...]) + bp2[...]


@functools.lru_cache(maxsize=None)
def _make_tc(n, d, h, final):
    blk = 1000
    grid = n // blk
    row = lambda i: (i, 0)
    rep = lambda i: (0, 0)
    mat = pl.BlockSpec((blk, d), row)
    w_l = pl.BlockSpec((h, d), rep)
    b_l = pl.BlockSpec((1, h), rep)
    in_specs = [mat, pl.BlockSpec((blk, 1), row), mat, w_l, b_l, w_l]
    if final:
        in_specs += [pl.BlockSpec((h, h), rep), b_l,
                     pl.BlockSpec((h, h), rep), b_l]
    return pl.pallas_call(
        _tc_final_body if final else _tc_layer_body,
        grid=(grid,),
        in_specs=in_specs,
        out_specs=pl.BlockSpec((blk, h), row),
        out_shape=jax.ShapeDtypeStruct((n, h), jnp.float32),
    )


def kernel(x, edge_index, Wl1, bl1, Wr1, Wl2, bl2, Wr2, Wp1, bp1, Wp2, bp2):
    n, d = x.shape
    h = Wl1.shape[0]
    src = edge_index[0]
    dst = edge_index[1]

    srcx, dstp, ch = _edge_setup(src, dst, n)
    dh = d // NC
    zrows = 128 * (-(-n // (128 * NS)))
    z2d = jnp.zeros((zrows, dh), jnp.float32)
    z1d = jnp.zeros((zrows,), jnp.float32)
    ones_h = jnp.ones((K,), jnp.float32)

    sc_agg1 = _make_sc_agg(n, d, ch, True)
    sc_agg2 = _make_sc_agg(n, h, ch, False)
    tc1 = _make_tc(n, d, h, False)
    tc2 = _make_tc(n, h, h, True)

    x2 = x.reshape(n * NC, dh)
    agg1, cntp = sc_agg1(x2, srcx, dstp, z2d, z1d, ones_h)
    cnt = cntp[:n].reshape(n, 1)

    bl1r = bl1.reshape(1, h)
    h1 = tc1(agg1, cnt, x, Wl1, bl1r, Wr1)

    h1_2 = h1.reshape(n * NC, h // NC)
    (agg2,) = sc_agg2(h1_2, srcx, dstp, z2d, z1d, ones_h)

    out = tc2(agg2, cnt, h1, Wl2, bl2.reshape(1, h), Wr2,
              Wp1, bp1.reshape(1, h), Wp2, bp2.reshape(1, h))
    return out
